# baseline (speedup 1.0000x reference)
import functools

import jax
import jax.numpy as jnp
from jax import lax
from jax.experimental import pallas as pl
from jax.experimental.pallas import tpu as pltpu
from jax.experimental.pallas import tpu_sc as plsc

_BATCH = 128
_POOL_SIZE = 2048
_MAX_CTX = 16384
_DRAFT = 8
_NUM_CORES = 2
_NUM_SUBCORES = 16
_NW = _NUM_CORES * _NUM_SUBCORES
_OUT = _BATCH * _DRAFT
_REQ_PER_W = _BATCH // _NW
_ELEM_PER_W = _REQ_PER_W * _DRAFT
_LANES = 16
_COMB_PAD = 2 * _BATCH + _LANES
_WIN = 256
_COL_MAX = _MAX_CTX - _WIN


def _sc_gather(comb_hbm, table_hbm):
    mesh = plsc.VectorSubcoreMesh(core_axis_name="c", subcore_axis_name="s")

    @functools.partial(
        pl.kernel,
        mesh=mesh,
        out_type=jax.ShapeDtypeStruct((_OUT,), jnp.int32),
        compiler_params=pltpu.CompilerParams(needs_layout_passes=False),
        scratch_types=[
            pltpu.VMEM((_LANES,), jnp.int32),
            pltpu.VMEM((_REQ_PER_W, 8, _WIN), jnp.int32),
            pltpu.VMEM((_ELEM_PER_W,), jnp.int32),
            pltpu.SemaphoreType.DMA,
        ],
    )
    def k(comb, table, out, comb_v, win_v, vals_v, sem):
        wid = lax.axis_index("s") * _NUM_CORES + lax.axis_index("c")
        base_out = wid * _ELEM_PER_W
        stage = pl.multiple_of(wid * 2 * _REQ_PER_W, 8)
        pltpu.async_copy(comb.at[pl.ds(stage, _LANES)], comb_v, sem).wait()
        lanes = lax.iota(jnp.int32, _LANES)
        cv = comb_v[pl.ds(0, _LANES)]
        copies = []
        for q in range(_REQ_PER_W):
            row = jnp.max(jnp.where(lanes == 2 * q, cv, 0))
            col = jnp.max(jnp.where(lanes == 2 * q + 1, cv, 0))
            row_a = pl.multiple_of(row & -8, 8)
            col_a = pl.multiple_of(jnp.minimum(col & -128, _COL_MAX), 128)
            copies.append(pltpu.async_copy(
                table.at[pl.ds(row_a, 8), pl.ds(col_a, _WIN)],
                win_v.at[jnp.int32(q)],
                sem))
        for cp in copies:
            cp.wait()
        for c in range(_ELEM_PER_W // _LANES):
            e = c * _LANES + lanes
            q_idx = e >> 3
            j = e & 7
            r_lane = plsc.load_gather(comb_v, [2 * q_idx])
            s_lane = plsc.load_gather(comb_v, [2 * q_idx + 1])
            col_a_lane = jnp.minimum(s_lane & -128, _COL_MAX)
            vals = plsc.load_gather(
                win_v, [q_idx, r_lane & 7, s_lane - col_a_lane + j])
            vals_v[pl.ds(c * _LANES, _LANES)] = vals
        pltpu.sync_copy(vals_v, out.at[pl.ds(base_out, _ELEM_PER_W)])

    return k(comb_hbm, table_hbm)


def kernel(req_pool_indices, req_to_token, start_offset, end_offset,
           batch_size, draft_token_num):
    del end_offset, batch_size, draft_token_num
    comb = jnp.stack(
        [req_pool_indices.astype(jnp.int32), start_offset.astype(jnp.int32)],
        axis=1,
    ).reshape(2 * _BATCH)
    comb = jnp.concatenate(
        [comb, jnp.zeros((_COMB_PAD - 2 * _BATCH,), jnp.int32)])
    return _sc_gather(comb, req_to_token)


# speedup vs baseline: 1.0141x; 1.0141x over previous
import functools

import jax
import jax.numpy as jnp
from jax import lax
from jax.experimental import pallas as pl
from jax.experimental.pallas import tpu as pltpu
from jax.experimental.pallas import tpu_sc as plsc

_BATCH = 128
_POOL_SIZE = 2048
_MAX_CTX = 16384
_DRAFT = 8
_NUM_CORES = 2
_NUM_SUBCORES = 16
_NW = _NUM_CORES * _NUM_SUBCORES
_OUT = _BATCH * _DRAFT
_REQ_PER_W = _BATCH // _NW
_ELEM_PER_W = _REQ_PER_W * _DRAFT
_PAD = _BATCH + 16
_LANES = 16
_WIN = 256
_COL_MAX = _MAX_CTX - _WIN


def _sc_gather(rpi_hbm, start_hbm, table_hbm):
    mesh = plsc.VectorSubcoreMesh(core_axis_name="c", subcore_axis_name="s")

    @functools.partial(
        pl.kernel,
        mesh=mesh,
        out_type=jax.ShapeDtypeStruct((_OUT,), jnp.int32),
        compiler_params=pltpu.CompilerParams(needs_layout_passes=False),
        scratch_types=[
            pltpu.VMEM((_LANES,), jnp.int32),
            pltpu.VMEM((_LANES,), jnp.int32),
            pltpu.VMEM((_REQ_PER_W, 2, 8, 128), jnp.int32),
            pltpu.VMEM((_ELEM_PER_W,), jnp.int32),
            pltpu.SemaphoreType.DMA,
            pltpu.SemaphoreType.DMA,
        ],
    )
    def k(rpi, start, table, out, rpi_v, start_v, win_v, vals_v, sem, sem2):
        wid = lax.axis_index("s") * _NUM_CORES + lax.axis_index("c")
        base_req = wid * _REQ_PER_W
        base_out = wid * _ELEM_PER_W
        stage = pl.multiple_of(
            jnp.minimum(base_req & -8, _BATCH - _LANES), 8)
        off = base_req - stage
        cp_r = pltpu.async_copy(rpi.at[pl.ds(stage, _LANES)], rpi_v, sem)
        cp_s = pltpu.async_copy(start.at[pl.ds(stage, _LANES)], start_v, sem)
        cp_r.wait()
        cp_s.wait()
        lanes = lax.iota(jnp.int32, _LANES)
        rchunk = plsc.load_gather(rpi_v, [off + (lanes & 3)])
        schunk = plsc.load_gather(start_v, [off + (lanes & 3)])
        copies = []
        crosses = []
        for q in range(_REQ_PER_W):
            row = jnp.max(jnp.where(lanes == q, rchunk, 0))
            col = jnp.max(jnp.where(lanes == q, schunk, 0))
            row_a = pl.multiple_of(row & -8, 8)
            col_a = pl.multiple_of(col & -128, 128)
            copies.append(pltpu.async_copy(
                table.at[pl.ds(row_a, 8), pl.ds(col_a, 128)],
                win_v.at[jnp.int32(q), jnp.int32(0)],
                sem))
            cross = (col & 127) > 120
            crosses.append(cross)
            col_b = pl.multiple_of(jnp.minimum(col_a + 128, _MAX_CTX - 128),
                                   128)

            @pl.when(cross)
            def _(row_a=row_a, col_b=col_b, q=q):
                pltpu.async_copy(
                    table.at[pl.ds(row_a, 8), pl.ds(col_b, 128)],
                    win_v.at[jnp.int32(q), jnp.int32(1)],
                    sem2)

        for q, (cp, cross) in enumerate(zip(copies, crosses)):
            cp.wait()

            @pl.when(cross)
            def _(q=q):
                pltpu.make_async_copy(
                    table.at[pl.ds(0, 8), pl.ds(0, 128)],
                    win_v.at[jnp.int32(q), jnp.int32(1)],
                    sem2).wait()

        for c in range(_ELEM_PER_W // _LANES):
            e = c * _LANES + lanes
            q_idx = e >> 3
            j = e & 7
            r_lane = plsc.load_gather(rpi_v, [off + q_idx])
            s_lane = plsc.load_gather(start_v, [off + q_idx])
            cwin = (s_lane & 127) + j
            vals = plsc.load_gather(
                win_v, [q_idx, cwin >> 7, r_lane & 7, cwin & 127])
            vals_v[pl.ds(c * _LANES, _LANES)] = vals
        pltpu.sync_copy(vals_v, out.at[pl.ds(base_out, _ELEM_PER_W)])

    return k(rpi_hbm, start_hbm, table_hbm)


def kernel(req_pool_indices, req_to_token, start_offset, end_offset,
           batch_size, draft_token_num):
    del end_offset, batch_size, draft_token_num
    rpi = req_pool_indices.astype(jnp.int32)
    start = start_offset.astype(jnp.int32)
    return _sc_gather(rpi, start, req_to_token)
